# baseline (speedup 1.0000x reference)
import functools

import jax
import jax.numpy as jnp
from jax import lax
from jax.experimental import pallas as pl
from jax.experimental.pallas import tpu as pltpu
from jax.experimental.pallas import tpu_sc as plsc

D_LANES = 16


def _pos_encoding(seq_len, d_model):
    pos = jnp.arange(seq_len, dtype=jnp.float32)[:, None]
    i = jnp.arange(0, d_model, 2, dtype=jnp.float32)
    angle = pos / jnp.power(10000.0, i / d_model)
    pe = jnp.zeros((seq_len, d_model), dtype=jnp.float32)
    pe = pe.at[:, 0::2].set(jnp.sin(angle))
    pe = pe.at[:, 1::2].set(jnp.cos(angle))
    return pe


def _make_sc_kernel(B, S, D, V, s_per_w, chunk):
    n_chunks = s_per_w // chunk
    vecs_per_row = D // D_LANES
    mesh = plsc.VectorSubcoreMesh(core_axis_name="c", subcore_axis_name="s")
    info = plsc.get_sparse_core_info()
    nc = info.num_cores

    @functools.partial(
        pl.kernel,
        mesh=mesh,
        out_type=jax.ShapeDtypeStruct((B * S, D), jnp.float32),
        scratch_types=[
            pltpu.VMEM((chunk,), jnp.int32),
            pltpu.VMEM((chunk, D), jnp.float32),
            pltpu.VMEM((chunk, D), jnp.float32),
            pltpu.SemaphoreType.DMA,
        ],
    )
    def k(x_hbm, table_hbm, pos_hbm, out_hbm, idx_v, pos_v, rows_v, sem):
        wid = lax.axis_index("s") * nc + lax.axis_index("c")
        w_s0 = wid * s_per_w
        for kc in range(n_chunks):
            s0 = w_s0 + kc * chunk
            pltpu.sync_copy(pos_hbm.at[pl.ds(s0, chunk)], pos_v)
            for b in range(B):
                base = b * S + s0
                pltpu.sync_copy(x_hbm.at[pl.ds(base, chunk)], idx_v)
                pltpu.async_copy(table_hbm.at[idx_v], rows_v, sem).wait()

                def body(r, _):
                    for j in range(vecs_per_row):
                        sl = pl.ds(j * D_LANES, D_LANES)
                        plsc.addupdate(rows_v.at[r, sl], pos_v[r, sl])
                    return 0

                lax.fori_loop(0, chunk, body, 0)
                pltpu.sync_copy(rows_v, out_hbm.at[pl.ds(base, chunk)])

    return k


def kernel(x, token_table):
    B, S = x.shape
    V, D = token_table.shape
    n_workers = 32
    s_per_w = S // n_workers
    pos = _pos_encoding(S, D)
    x_flat = x.reshape(B * S)
    k = _make_sc_kernel(B, S, D, V, s_per_w, chunk=32)
    out = k(x_flat, token_table, pos)
    return out.reshape(B, S, D)


# speedup vs baseline: 1.1315x; 1.1315x over previous
import functools

import jax
import jax.numpy as jnp
from jax import lax
from jax.experimental import pallas as pl
from jax.experimental.pallas import tpu as pltpu
from jax.experimental.pallas import tpu_sc as plsc

D_LANES = 16


def _pos_encoding(seq_len, d_model):
    pos = jnp.arange(seq_len, dtype=jnp.float32)[:, None]
    i = jnp.arange(0, d_model, 2, dtype=jnp.float32)
    angle = pos / jnp.power(10000.0, i / d_model)
    pe = jnp.zeros((seq_len, d_model), dtype=jnp.float32)
    pe = pe.at[:, 0::2].set(jnp.sin(angle))
    pe = pe.at[:, 1::2].set(jnp.cos(angle))
    return pe


def _make_sc_kernel(B, S, D, s_per_w, chunk):
    n_chunks = s_per_w // chunk
    n_tasks = n_chunks * B
    vecs_per_row = D // D_LANES
    mesh = plsc.VectorSubcoreMesh(core_axis_name="c", subcore_axis_name="s")
    info = plsc.get_sparse_core_info()
    nc = info.num_cores

    @functools.partial(
        pl.kernel,
        mesh=mesh,
        out_type=jax.ShapeDtypeStruct((B * S, D), jnp.float32),
        scratch_types=[
            pltpu.VMEM((B, s_per_w), jnp.int32),
            pltpu.VMEM((chunk, D), jnp.float32),
            pltpu.VMEM((2, chunk, D), jnp.float32),
            pltpu.SemaphoreType.DMA,
            pltpu.SemaphoreType.DMA,
            pltpu.SemaphoreType.DMA,
            pltpu.SemaphoreType.DMA,
        ],
    )
    def k(x_hbm, table_hbm, pos_hbm, out_hbm, idx_all, pos_v, rows, g0, g1, s0_, s1_):
        wid = lax.axis_index("s") * nc + lax.axis_index("c")
        w_s0 = wid * s_per_w
        gsem = [g0, g1]
        ssem = [s0_, s1_]

        for b in range(B):
            pltpu.sync_copy(x_hbm.at[pl.ds(b * S + w_s0, s_per_w)], idx_all.at[b])

        def idx_ref(t):
            kc, b = divmod(t, B)
            return idx_all.at[b, pl.ds(kc * chunk, chunk)]

        gh = [None, None]
        sh = [None, None]
        gh[0] = pltpu.async_copy(table_hbm.at[idx_ref(0)], rows.at[0], gsem[0])
        for t in range(n_tasks):
            p = t & 1
            kc, b = divmod(t, B)
            if t + 1 < n_tasks:
                pn = (t + 1) & 1
                if sh[pn] is not None:
                    sh[pn].wait()
                gh[pn] = pltpu.async_copy(
                    table_hbm.at[idx_ref(t + 1)], rows.at[pn], gsem[pn])
            if b == 0:
                pltpu.sync_copy(
                    pos_hbm.at[pl.ds(w_s0 + kc * chunk, chunk)], pos_v)
            gh[p].wait()

            def body(r, _):
                for j in range(vecs_per_row):
                    sl = pl.ds(j * D_LANES, D_LANES)
                    plsc.addupdate(rows.at[p, r, sl], pos_v[r, sl])
                return 0

            lax.fori_loop(0, chunk, body, 0)
            sh[p] = pltpu.async_copy(
                rows.at[p],
                out_hbm.at[pl.ds(b * S + w_s0 + kc * chunk, chunk)],
                ssem[p])
        for h in sh:
            if h is not None:
                h.wait()

    return k


def kernel(x, token_table):
    B, S = x.shape
    V, D = token_table.shape
    n_workers = 32
    s_per_w = S // n_workers
    pos = _pos_encoding(S, D)
    x_flat = x.reshape(B * S)
    k = _make_sc_kernel(B, S, D, s_per_w, chunk=32)
    out = k(x_flat, token_table, pos)
    return out.reshape(B, S, D)


# speedup vs baseline: 1.9260x; 1.7022x over previous
import functools

import jax
import jax.numpy as jnp
import numpy as np
from jax import lax
from jax.experimental import pallas as pl
from jax.experimental.pallas import tpu as pltpu
from jax.experimental.pallas import tpu_sc as plsc

D_LANES = 16


def _pos_encoding(seq_len, d_model):
    pos = np.arange(seq_len, dtype=np.float32)[:, None]
    i = np.arange(0, d_model, 2, dtype=np.float32)
    angle = (pos / np.power(np.float32(10000.0), i / np.float32(d_model))
             ).astype(np.float32)
    pe = np.zeros((seq_len, d_model), dtype=np.float32)
    pe[:, 0::2] = np.sin(angle)
    pe[:, 1::2] = np.cos(angle)
    return jnp.asarray(pe)


def _make_sc_kernel(B, S, D, s_per_w, chunk):
    n_chunks = s_per_w // chunk
    n_tasks = n_chunks * B
    vecs_per_row = D // D_LANES
    mesh = plsc.VectorSubcoreMesh(core_axis_name="c", subcore_axis_name="s")
    info = plsc.get_sparse_core_info()
    nc = info.num_cores

    @functools.partial(
        pl.kernel,
        mesh=mesh,
        out_type=jax.ShapeDtypeStruct((B * S, D), jnp.float32),
        scratch_types=[
            pltpu.VMEM((B, s_per_w), jnp.int32),
            pltpu.VMEM((chunk, D), jnp.float32),
            pltpu.VMEM((2, chunk, D), jnp.float32),
            pltpu.SemaphoreType.DMA,
            pltpu.SemaphoreType.DMA,
            pltpu.SemaphoreType.DMA,
            pltpu.SemaphoreType.DMA,
        ],
    )
    def k(x_hbm, table_hbm, pos_hbm, out_hbm, idx_all, pos_v, rows, g0, g1, s0_, s1_):
        wid = lax.axis_index("s") * nc + lax.axis_index("c")
        w_s0 = wid * s_per_w
        gsem = [g0, g1]
        ssem = [s0_, s1_]

        for b in range(B):
            pltpu.sync_copy(x_hbm.at[pl.ds(b * S + w_s0, s_per_w)], idx_all.at[b])

        def idx_ref(t):
            kc, b = divmod(t, B)
            return idx_all.at[b, pl.ds(kc * chunk, chunk)]

        gh = [None, None]
        sh = [None, None]
        gh[0] = pltpu.async_copy(table_hbm.at[idx_ref(0)], rows.at[0], gsem[0])
        for t in range(n_tasks):
            p = t & 1
            kc, b = divmod(t, B)
            if t + 1 < n_tasks:
                pn = (t + 1) & 1
                if sh[pn] is not None:
                    sh[pn].wait()
                gh[pn] = pltpu.async_copy(
                    table_hbm.at[idx_ref(t + 1)], rows.at[pn], gsem[pn])
            if b == 0:
                pltpu.sync_copy(
                    pos_hbm.at[pl.ds(w_s0 + kc * chunk, chunk)], pos_v)
            gh[p].wait()

            def body(r, _):
                for j in range(vecs_per_row):
                    sl = pl.ds(j * D_LANES, D_LANES)
                    plsc.addupdate(rows.at[p, r, sl], pos_v[r, sl])
                return 0

            lax.fori_loop(0, chunk, body, 0)
            sh[p] = pltpu.async_copy(
                rows.at[p],
                out_hbm.at[pl.ds(b * S + w_s0 + kc * chunk, chunk)],
                ssem[p])
        for h in sh:
            if h is not None:
                h.wait()

    return k


def kernel(x, token_table):
    B, S = x.shape
    V, D = token_table.shape
    n_workers = 32
    s_per_w = S // n_workers
    pos = _pos_encoding(S, D)
    x_flat = x.reshape(B * S)
    k = _make_sc_kernel(B, S, D, s_per_w, chunk=32)
    out = k(x_flat, token_table, pos)
    return out.reshape(B, S, D)


# speedup vs baseline: 2.1981x; 1.1413x over previous
import functools

import jax
import jax.numpy as jnp
import numpy as np
from jax import lax
from jax.experimental import pallas as pl
from jax.experimental.pallas import tpu as pltpu
from jax.experimental.pallas import tpu_sc as plsc

D_LANES = 16


def _pos_encoding(seq_len, d_model):
    pos = np.arange(seq_len, dtype=np.float32)[:, None]
    i = np.arange(0, d_model, 2, dtype=np.float32)
    angle = (pos / np.power(np.float32(10000.0), i / np.float32(d_model))
             ).astype(np.float32)
    pe = np.zeros((seq_len, d_model), dtype=np.float32)
    pe[:, 0::2] = np.sin(angle)
    pe[:, 1::2] = np.cos(angle)
    return jnp.asarray(pe)


def _make_sc_kernel(B, S, D, s_per_w, chunk, nbuf=4):
    n_chunks = s_per_w // chunk
    n_tasks = n_chunks * B
    vecs_per_row = D // D_LANES
    mesh = plsc.VectorSubcoreMesh(core_axis_name="c", subcore_axis_name="s")
    info = plsc.get_sparse_core_info()
    nc = info.num_cores

    assert nbuf == B, "slot = batch index requires nbuf == B"
    n_pairs = (n_chunks - 2) // 2

    @functools.partial(
        pl.kernel,
        mesh=mesh,
        out_type=jax.ShapeDtypeStruct((B * S, D), jnp.float32),
        scratch_types=[
            pltpu.VMEM((B, s_per_w), jnp.int32),
            pltpu.VMEM((2, chunk, D), jnp.float32),
            pltpu.VMEM((nbuf, chunk, D), jnp.float32),
            pltpu.SemaphoreType.DMA,
            pltpu.SemaphoreType.DMA,
        ] + [pltpu.SemaphoreType.DMA] * (2 * nbuf),
    )
    def k(x_hbm, table_hbm, pos_hbm, out_hbm, idx_all, pos_v, rows, p0, p1, *sems):
        wid = lax.axis_index("s") * nc + lax.axis_index("c")
        w_s0 = wid * s_per_w
        psem = [p0, p1]
        gsem = list(sems[:nbuf])
        ssem = list(sems[nbuf:])

        for b in range(B):
            pltpu.sync_copy(x_hbm.at[pl.ds(b * S + w_s0, s_per_w)], idx_all.at[b])

        def gather_issue(kc, b):
            pltpu.async_copy(
                table_hbm.at[idx_all.at[b, pl.ds(kc * chunk, chunk)]],
                rows.at[b], gsem[b])

        def gather_wait(b):
            pltpu.make_async_copy(
                table_hbm.at[pl.ds(0, chunk)], rows.at[b], gsem[b]).wait()

        def store_issue(kc, b):
            pltpu.async_copy(
                rows.at[b],
                out_hbm.at[pl.ds(b * S + w_s0 + kc * chunk, chunk)], ssem[b])

        def store_wait(b):
            pltpu.make_async_copy(
                rows.at[b], out_hbm.at[pl.ds(0, chunk)], ssem[b]).wait()

        def pos_issue(kc, par):
            pltpu.async_copy(
                pos_hbm.at[pl.ds(w_s0 + kc * chunk, chunk)],
                pos_v.at[par], psem[par])

        def pos_wait(par):
            pltpu.make_async_copy(
                pos_hbm.at[pl.ds(0, chunk)], pos_v.at[par], psem[par]).wait()

        def process(kc, b, par, skip_swait=False, gnext=None):
            if not skip_swait:
                store_wait((b + 2) % nbuf)
            if gnext is not None:
                gather_issue(*gnext)
            gather_wait(b)
            pv = pos_v.at[par]

            def body(r, _):
                for j2 in range(vecs_per_row):
                    sl = pl.ds(j2 * D_LANES, D_LANES)
                    plsc.addupdate(rows.at[b, r, sl], pv[r, sl])
                return 0

            lax.fori_loop(0, chunk, body, 0)
            store_issue(kc, b)

        pos_issue(0, 0)
        gather_issue(0, 0)
        gather_issue(0, 1)
        pos_wait(0)
        pos_issue(1, 1)
        process(0, 0, 0, skip_swait=True, gnext=(0, 2))
        process(0, 1, 0, skip_swait=True, gnext=(0, 3))
        process(0, 2, 0, gnext=(1, 0))
        process(0, 3, 0, gnext=(1, 1))

        def pair_body(p, _):
            kc = 1 + 2 * p
            pos_wait(1)
            pos_issue(kc + 1, 0)
            process(kc, 0, 1, gnext=(kc, 2))
            process(kc, 1, 1, gnext=(kc, 3))
            process(kc, 2, 1, gnext=(kc + 1, 0))
            process(kc, 3, 1, gnext=(kc + 1, 1))
            pos_wait(0)
            pos_issue(kc + 2, 1)
            process(kc + 1, 0, 0, gnext=(kc + 1, 2))
            process(kc + 1, 1, 0, gnext=(kc + 1, 3))
            process(kc + 1, 2, 0, gnext=(kc + 2, 0))
            process(kc + 1, 3, 0, gnext=(kc + 2, 1))
            return 0

        lax.fori_loop(0, n_pairs, pair_body, 0)

        last = n_chunks - 1
        pos_wait(last & 1)
        process(last, 0, last & 1, gnext=(last, 2))
        process(last, 1, last & 1, gnext=(last, 3))
        process(last, 2, last & 1)
        process(last, 3, last & 1)
        store_wait(2)
        store_wait(3)

    return k


def kernel(x, token_table):
    B, S = x.shape
    V, D = token_table.shape
    n_workers = 32
    s_per_w = S // n_workers
    pos = _pos_encoding(S, D)
    x_flat = x.reshape(B * S)
    k = _make_sc_kernel(B, S, D, s_per_w, chunk=16)
    out = k(x_flat, token_table, pos)
    return out.reshape(B, S, D)
